# Initial kernel scaffold; baseline (speedup 1.0000x reference)
import jax
import jax.numpy as jnp
from jax.experimental import pallas as pl


def kernel(x, adj, W1, b1, Wx, bx, W2, b2):
    raise NotImplementedError("write your pallas kernel here")


# speedup vs baseline: 16.9111x; 16.9111x over previous
import functools

import jax
import jax.numpy as jnp
from jax import lax
from jax.experimental import pallas as pl
from jax.experimental.pallas import tpu as pltpu
from jax.experimental.pallas import tpu_sc as plsc

N = 10000
E = 320000
NC = 2
NS = 16
EPW = E // (NC * NS)
EK = 200
NPAD = 10240
ROWS_PER_TILE = NPAD // NS
RCHUNK = 128
DEG_PAD = 10240
DEG_CHUNK = DEG_PAD // NS

_MESH = plsc.VectorSubcoreMesh(core_axis_name="c", subcore_axis_name="s")


@functools.partial(
    pl.kernel,
    out_type=jax.ShapeDtypeStruct((NC * DEG_PAD,), jnp.float32),
    mesh=_MESH,
    scratch_types=[
        pltpu.VMEM((EK,), jnp.int32),
        pltpu.VMEM((EK,), jnp.float32),
        pltpu.VMEM((DEG_CHUNK,), jnp.float32),
        pltpu.VMEM_SHARED((DEG_PAD,), jnp.float32),
    ],
)
def _deg_kernel(dst_hbm, ones_hbm, zeros_hbm, out_hbm, didx, ones_v, zb, shared):
    c = lax.axis_index("c")
    s = lax.axis_index("s")
    pltpu.sync_copy(zeros_hbm, zb)
    pltpu.sync_copy(zb, shared.at[pl.ds(s * DEG_CHUNK, DEG_CHUNK)])
    pltpu.sync_copy(ones_hbm, ones_v)
    plsc.subcore_barrier()
    base = (c * NS + s) * EPW

    def body(i, carry):
        pltpu.sync_copy(dst_hbm.at[pl.ds(base + i * EK, EK)], didx)
        pltpu.sync_copy(ones_v, shared.at[didx], add=True)
        return carry

    lax.fori_loop(0, EPW // EK, body, 0)
    plsc.subcore_barrier()
    pltpu.sync_copy(shared.at[pl.ds(s * DEG_CHUNK, DEG_CHUNK)], zb)
    pltpu.sync_copy(zb, out_hbm.at[pl.ds(c * DEG_PAD + s * DEG_CHUNK, DEG_CHUNK)])


def _make_agg_kernel(d):
    @functools.partial(
        pl.kernel,
        out_type=jax.ShapeDtypeStruct((NC, NPAD, d), jnp.float32),
        mesh=_MESH,
        scratch_types=[
            pltpu.VMEM((EK,), jnp.int32),
            pltpu.VMEM((EK,), jnp.int32),
            pltpu.VMEM((EK, d), jnp.float32),
            pltpu.VMEM((RCHUNK, d), jnp.float32),
            pltpu.VMEM_SHARED((NPAD, d), jnp.float32),
            pltpu.SemaphoreType.DMA,
        ],
    )
    def agg(g_hbm, src_hbm, dst_hbm, zeros_hbm, out_hbm,
            sidx, didx, rows, zb, shared, sem):
        c = lax.axis_index("c")
        s = lax.axis_index("s")
        pltpu.sync_copy(zeros_hbm, zb)
        for j in range(ROWS_PER_TILE // RCHUNK):
            pltpu.sync_copy(
                zb, shared.at[pl.ds(s * ROWS_PER_TILE + j * RCHUNK, RCHUNK)])
        plsc.subcore_barrier()
        base = (c * NS + s) * EPW

        def body(i, carry):
            e = base + i * EK
            pltpu.sync_copy(src_hbm.at[pl.ds(e, EK)], sidx)
            pltpu.sync_copy(dst_hbm.at[pl.ds(e, EK)], didx)
            pltpu.async_copy(g_hbm.at[sidx], rows, sem).wait()
            pltpu.sync_copy(rows, shared.at[didx], add=True)
            return carry

        lax.fori_loop(0, EPW // EK, body, 0)
        plsc.subcore_barrier()
        for j in range(ROWS_PER_TILE // RCHUNK):
            r = s * ROWS_PER_TILE + j * RCHUNK
            pltpu.sync_copy(shared.at[pl.ds(r, RCHUNK)], zb)
            pltpu.sync_copy(zb, out_hbm.at[c, pl.ds(r, RCHUNK)])

    return agg


_agg128 = _make_agg_kernel(128)


BM = 2000


def _dinv_body(deg_ref, out_ref):
    d = deg_ref[:, 0:1] + deg_ref[:, 1:2] + 1.0
    out_ref[...] = lax.rsqrt(d)


def _dinv(deg_t):
    return pl.pallas_call(
        _dinv_body,
        grid=(N // BM,),
        in_specs=[pl.BlockSpec((BM, 2), lambda i: (i, 0))],
        out_specs=pl.BlockSpec((BM, 1), lambda i: (i, 0)),
        out_shape=jax.ShapeDtypeStruct((N, 1), jnp.float32),
    )(deg_t)


def _mm1_body(x_ref, w_ref, dinv_ref, out_ref):
    h = jnp.dot(x_ref[...], w_ref[...], preferred_element_type=jnp.float32)
    out_ref[...] = dinv_ref[...] * h


def _mm1(x, w, dinv):
    k, n_out = w.shape
    return pl.pallas_call(
        _mm1_body,
        grid=(N // BM,),
        in_specs=[
            pl.BlockSpec((BM, k), lambda i: (i, 0)),
            pl.BlockSpec((k, n_out), lambda i: (0, 0)),
            pl.BlockSpec((BM, 1), lambda i: (i, 0)),
        ],
        out_specs=pl.BlockSpec((BM, n_out), lambda i: (i, 0)),
        out_shape=jax.ShapeDtypeStruct((N, n_out), jnp.float32),
    )(x, w, dinv)


def _mm2_body(p_ref, g_ref, dinv_ref, b_ref, w_ref, out_ref):
    dinv = dinv_ref[...]
    agg = p_ref[0] + p_ref[1] + g_ref[...]
    h = jnp.maximum(dinv * agg + b_ref[...], 0.0)
    out_ref[...] = dinv * jnp.dot(h, w_ref[...],
                                  preferred_element_type=jnp.float32)


def _mm2(p, g, dinv, b, w):
    k, n_out = w.shape
    return pl.pallas_call(
        _mm2_body,
        grid=(N // BM,),
        in_specs=[
            pl.BlockSpec((2, BM, k), lambda i: (0, i, 0)),
            pl.BlockSpec((BM, k), lambda i: (i, 0)),
            pl.BlockSpec((BM, 1), lambda i: (i, 0)),
            pl.BlockSpec((1, k), lambda i: (0, 0)),
            pl.BlockSpec((k, n_out), lambda i: (0, 0)),
        ],
        out_specs=pl.BlockSpec((BM, n_out), lambda i: (i, 0)),
        out_shape=jax.ShapeDtypeStruct((N, n_out), jnp.float32),
    )(p, g, dinv, b, w)


def _fin_body(p_ref, g_ref, dinv_ref, b_ref, out_ref):
    d = out_ref.shape[1]
    agg = p_ref[0, :, :d] + p_ref[1, :, :d] + g_ref[:, :d]
    out_ref[...] = dinv_ref[...] * agg + b_ref[...]


def _fin(p, g, dinv, b):
    d = b.shape[1]
    return pl.pallas_call(
        _fin_body,
        grid=(N // BM,),
        in_specs=[
            pl.BlockSpec((2, BM, 128), lambda i: (0, i, 0)),
            pl.BlockSpec((BM, 128), lambda i: (i, 0)),
            pl.BlockSpec((BM, 1), lambda i: (i, 0)),
            pl.BlockSpec((1, d), lambda i: (0, 0)),
        ],
        out_specs=pl.BlockSpec((BM, d), lambda i: (i, 0)),
        out_shape=jax.ShapeDtypeStruct((N, d), jnp.float32),
    )(p, g, dinv, b)


def kernel(x, adj, W1, b1, Wx, bx, W2, b2):
    src = adj[0]
    dst = adj[1]

    ones_ek = jnp.ones((EK,), jnp.float32)
    zeros_deg = jnp.zeros((DEG_CHUNK,), jnp.float32)
    zeros128 = jnp.zeros((RCHUNK, 128), jnp.float32)
    W2p = jnp.pad(W2, ((0, 0), (0, 128 - W2.shape[1])))

    deg_flat = _deg_kernel(dst, ones_ek, zeros_deg)
    deg_pair = deg_flat.reshape(NC, DEG_PAD)
    deg_t = jnp.transpose(deg_pair[:, :N])
    dinv = _dinv(deg_t)

    g1 = _mm1(x, W1, dinv)
    p1 = _agg128(g1, src, dst, zeros128)
    g2 = _mm2(p1, g1, dinv, b1.reshape(1, -1), Wx)
    p2 = _agg128(g2, src, dst, zeros128)
    g3 = _mm2(p2, g2, dinv, bx.reshape(1, -1), W2p)
    p3 = _agg128(g3, src, dst, zeros128)
    return _fin(p3, g3, dinv, b2.reshape(1, -1))
